# baseline (speedup 1.0000x reference)
import jax
import jax.numpy as jnp
from jax import lax
from jax.experimental import pallas as pl
from jax.experimental.pallas import tpu as pltpu
from jax.experimental.pallas import tpu_sc as plsc

TOKEN_SEQ_LEN = 1024
W = 1536
N = 64 * 1024
NC, NS, L = 2, 16, 16
NW = NC * NS
RPW = N // NW
C = 16
NBUF = 4
NCH = RPW // C


def _sc_lookup(pos_hbm, mask_hbm, table_hbm, out_hbm,
               mask_v, idx_v, bufs, gsems, ssems):
    wid = lax.axis_index("s") * NC + lax.axis_index("c")
    base = wid * RPW

    pltpu.sync_copy(pos_hbm.at[pl.ds(base, RPW)], idx_v)
    pltpu.sync_copy(mask_hbm.at[pl.ds(base, RPW)], mask_v)

    def idx_body(i):
        sl = pl.ds(i * L, L)
        idx_v[sl] = jnp.where(mask_v[sl] != jnp.int32(0), idx_v[sl],
                              jnp.int32(TOKEN_SEQ_LEN))

    pl.loop(0, RPW // L)(idx_body)

    def gather(c, b):
        pltpu.async_copy(table_hbm.at[idx_v.at[pl.ds(c * C, C)]],
                         bufs[b], gsems[b])

    def scatter(c, b):
        pltpu.async_copy(bufs[b], out_hbm.at[pl.ds(base + c * C, C)],
                         ssems[b])

    def wait(sem, b):
        pltpu.make_async_copy(table_hbm.at[pl.ds(0, C)], bufs[b], sem).wait()

    def chunk_body(g):
        c0 = g * NBUF
        for b in range(NBUF):
            gather(c0 + b, b)
        for b in range(NBUF):
            wait(gsems[b], b)

    pl.loop(0, NCH // NBUF)(chunk_body)
    scatter(0, 0)
    wait(ssems[0], 0)


@jax.jit
def kernel(obs_pos, obs_mask, table):
    B, S = obs_pos.shape
    table_p = jnp.concatenate(
        [table, jnp.zeros((1, W), table.dtype)], axis=0)
    pos = obs_pos.reshape(N)
    mask = obs_mask.reshape(N)

    mesh = plsc.VectorSubcoreMesh(
        core_axis_name="c", subcore_axis_name="s",
        num_cores=NC, num_subcores=NS)
    out = pl.kernel(
        _sc_lookup,
        out_type=jax.ShapeDtypeStruct((N, W), jnp.float32),
        mesh=mesh,
        scratch_types=[
            pltpu.VMEM((RPW,), jnp.int32),
            pltpu.VMEM((RPW,), jnp.int32),
            [pltpu.VMEM((C, W), jnp.float32) for _ in range(NBUF)],
            [pltpu.SemaphoreType.DMA for _ in range(NBUF)],
            [pltpu.SemaphoreType.DMA for _ in range(NBUF)],
        ],
    )(pos, mask, table_p)
    return out.reshape(B, S, W)


# speedup vs baseline: 4.8415x; 4.8415x over previous
import jax
import jax.numpy as jnp
from jax import lax
from jax.experimental import pallas as pl
from jax.experimental.pallas import tpu as pltpu
from jax.experimental.pallas import tpu_sc as plsc

TOKEN_SEQ_LEN = 1024
W = 1536
N = 64 * 1024
NC, NS, L = 2, 16, 16
NW = NC * NS
TR = TOKEN_SEQ_LEN // NW
ZROW = TR
SEG = 2048
NSEG = N // SEG
DEPTH = 8


def _sc_push(pos_hbm, mask_hbm, table_hbm, out_hbm,
             tblbuf, pseg, mseg, olist, cntbuf, wsem):
    wid = lax.axis_index("s") * NC + lax.axis_index("c")
    r0 = wid * TR

    pltpu.sync_copy(table_hbm.at[pl.ds(r0, TR)], tblbuf.at[pl.ds(0, TR)])
    def zero_body(g):
        tblbuf[ZROW, pl.ds(g * L, L)] = jnp.zeros((L,), jnp.float32)

    pl.loop(0, W // L)(zero_body)

    def fire(j):
        v = olist[pl.ds(j, L)][0]
        o = v & jnp.int32(0xFFFF)
        rl = lax.shift_right_logical(v, jnp.int32(16))
        pltpu.async_copy(tblbuf.at[pl.ds(rl, 1)],
                         out_hbm.at[pl.ds(o, 1)], wsem)

    def drain_one():
        pltpu.make_async_copy(table_hbm.at[pl.ds(0, 1)],
                              tblbuf.at[pl.ds(0, 1)], wsem).wait()

    def seg_body(s, inflight):
        pltpu.sync_copy(pos_hbm.at[pl.ds(s * SEG, SEG)], pseg)
        pltpu.sync_copy(mask_hbm.at[pl.ds(s * SEG, SEG)], mseg)

        obase = s * SEG + lax.iota(jnp.int32, L)

        def scan_body(i, off):
            sl = pl.ds(i * L, L)
            p = pseg[sl]
            m = mseg[sl]
            rl = jnp.where(m != jnp.int32(0), p - r0, jnp.int32(ZROW))
            inr = plsc.bitcast(p - r0, jnp.uint32) < jnp.uint32(TR)
            val = (obase + i * L) | lax.shift_left(rl, jnp.int32(16))
            key = jnp.where(inr, jnp.uint32(0), jnp.uint32(1))
            _, vs = plsc.sort_key_val(key, val)
            olist[pl.ds(off, L)] = vs
            cntbuf[pl.ds(0, L)] = plsc.all_reduce_population_count(inr)
            return off + cntbuf[pl.ds(0, L)][0]

        cnt = lax.fori_loop(0, SEG // L, scan_body, jnp.int32(0))

        def fire_body(j, fly):
            fire(j)

            @pl.when(fly >= DEPTH)
            def _():
                drain_one()

            return jnp.minimum(fly + 1, DEPTH)

        return lax.fori_loop(0, cnt, fire_body, inflight)

    inflight = lax.fori_loop(0, NSEG, seg_body, jnp.int32(0))

    def tail_body(t):
        @pl.when(t < inflight)
        def _():
            drain_one()

    pl.loop(0, DEPTH)(tail_body)


@jax.jit
def kernel(obs_pos, obs_mask, table):
    B, S = obs_pos.shape
    pos = obs_pos.reshape(N)
    mask = obs_mask.reshape(N)

    mesh = plsc.VectorSubcoreMesh(
        core_axis_name="c", subcore_axis_name="s",
        num_cores=NC, num_subcores=NS)
    out = pl.kernel(
        _sc_push,
        out_type=jax.ShapeDtypeStruct((N, W), jnp.float32),
        mesh=mesh,
        compiler_params=pltpu.CompilerParams(needs_layout_passes=False),
        scratch_types=[
            pltpu.VMEM((TR + 8, W), jnp.float32),
            pltpu.VMEM((SEG,), jnp.int32),
            pltpu.VMEM((SEG,), jnp.int32),
            pltpu.VMEM((SEG + L,), jnp.int32),
            pltpu.VMEM((L,), jnp.int32),
            pltpu.SemaphoreType.DMA,
        ],
    )(pos, mask, table)
    return out.reshape(B, S, W)


# speedup vs baseline: 4.8917x; 1.0104x over previous
import jax
import jax.numpy as jnp
from jax import lax
from jax.experimental import pallas as pl
from jax.experimental.pallas import tpu as pltpu
from jax.experimental.pallas import tpu_sc as plsc

TOKEN_SEQ_LEN = 1024
W = 1536
N = 64 * 1024
NC, NS, L = 2, 16, 16
NW = NC * NS
TR = TOKEN_SEQ_LEN // NW
ZROW = TR
SEG = 2048
NSEG = N // SEG
DEPTH = 16


def _sc_push(pos_hbm, mask_hbm, table_hbm, out_hbm,
             tblbuf, pseg, mseg, olist, cntbuf, wsem):
    wid = lax.axis_index("s") * NC + lax.axis_index("c")
    r0 = wid * TR

    pltpu.sync_copy(table_hbm.at[pl.ds(r0, TR)], tblbuf.at[pl.ds(0, TR)])
    def zero_body(g):
        tblbuf[ZROW, pl.ds(g * L, L)] = jnp.zeros((L,), jnp.float32)

    pl.loop(0, W // L)(zero_body)

    def fire(j):
        v = olist[pl.ds(j, L)][0]
        o = v & jnp.int32(0xFFFF)
        rl = lax.shift_right_logical(v, jnp.int32(16))
        pltpu.async_copy(tblbuf.at[pl.ds(rl, 1)],
                         out_hbm.at[pl.ds(o, 1)], wsem)

    def drain_one():
        pltpu.make_async_copy(table_hbm.at[pl.ds(0, 1)],
                              tblbuf.at[pl.ds(0, 1)], wsem).wait()

    def seg_body(s, inflight):
        pltpu.sync_copy(pos_hbm.at[pl.ds(s * SEG, SEG)], pseg)
        pltpu.sync_copy(mask_hbm.at[pl.ds(s * SEG, SEG)], mseg)

        lanes = lax.iota(jnp.int32, L)
        obase = s * SEG + lanes

        def scan_body(i, offv):
            sl = pl.ds(i * L, L)
            p = pseg[sl]
            m = mseg[sl]
            rl = jnp.where(m != jnp.int32(0), p - r0, jnp.int32(ZROW))
            inr = plsc.bitcast(p - r0, jnp.uint32) < jnp.uint32(TR)
            val = (obase + i * L) | lax.shift_left(rl, jnp.int32(16))
            key = jnp.where(inr, jnp.uint32(0), jnp.uint32(1))
            _, vs = plsc.sort_key_val(key, val)
            plsc.store_scatter(olist, [offv + lanes], vs)
            return offv + plsc.all_reduce_population_count(inr)

        offv = lax.fori_loop(0, SEG // L, scan_body,
                             jnp.zeros((L,), jnp.int32))
        cntbuf[pl.ds(0, L)] = offv
        cnt = cntbuf[pl.ds(0, L)][0]

        def fire_body(j, fly):
            fire(j)

            @pl.when(fly >= DEPTH)
            def _():
                drain_one()

            return jnp.minimum(fly + 1, DEPTH)

        return lax.fori_loop(0, cnt, fire_body, inflight)

    inflight = lax.fori_loop(0, NSEG, seg_body, jnp.int32(0))

    def tail_body(t):
        @pl.when(t < inflight)
        def _():
            drain_one()

    pl.loop(0, DEPTH)(tail_body)


@jax.jit
def kernel(obs_pos, obs_mask, table):
    B, S = obs_pos.shape
    pos = obs_pos.reshape(N)
    mask = obs_mask.reshape(N)

    mesh = plsc.VectorSubcoreMesh(
        core_axis_name="c", subcore_axis_name="s",
        num_cores=NC, num_subcores=NS)
    out = pl.kernel(
        _sc_push,
        out_type=jax.ShapeDtypeStruct((N, W), jnp.float32),
        mesh=mesh,
        compiler_params=pltpu.CompilerParams(needs_layout_passes=False),
        scratch_types=[
            pltpu.VMEM((TR + 8, W), jnp.float32),
            pltpu.VMEM((SEG,), jnp.int32),
            pltpu.VMEM((SEG,), jnp.int32),
            pltpu.VMEM((SEG + L,), jnp.int32),
            pltpu.VMEM((L,), jnp.int32),
            pltpu.SemaphoreType.DMA,
        ],
    )(pos, mask, table)
    return out.reshape(B, S, W)
